# baseline (speedup 1.0000x reference)
import functools

import jax
import jax.numpy as jnp
from jax import lax
from jax.experimental import pallas as pl
from jax.experimental.pallas import tpu as pltpu
from jax.experimental.pallas import tpu_sc as plsc

VOCAB = 1000000
EMB = 32
HIDDEN = 50
OUT = 100
SEQ = 200
BATCH = 4096

NC = 2
NS = 16
NW = NC * NS
BPW = BATCH // NW
NBUF = 4
NSTEP = SEQ // NBUF
ACC_UNROLL = 8

TBLK = 32768
SUB = TBLK // 4
NTBLK = (VOCAB + TBLK - 1) // TBLK
VPAD = NTBLK * TBLK
PK_ROWS = VPAD // 4


def _pack_table_tc(tT):
    def body(t_ref, o_ref):
        xblk = t_ref[...]
        xs = jnp.concatenate(
            [xblk[:, a * SUB:(a + 1) * SUB] for a in range(4)], axis=0)
        eye = jnp.eye(128, dtype=jnp.float32)
        packed = lax.dot_general(
            xs, eye, (((0,), (0,)), ((), ())),
            preferred_element_type=jnp.float32)
        o_ref[...] = packed.astype(jnp.bfloat16)

    return pl.pallas_call(
        body,
        grid=(NTBLK,),
        in_specs=[pl.BlockSpec((EMB, TBLK), lambda i: (0, i))],
        out_specs=pl.BlockSpec((TBLK // 4, 128), lambda i: (i, 0)),
        out_shape=jax.ShapeDtypeStruct((PK_ROWS, 128), jnp.bfloat16),
        compiler_params=pltpu.CompilerParams(fuse_transposed_lhs_in_matmul=True),
    )(tT)


def _pooled_sum_sc(xr, tab_lin):
    mesh = plsc.VectorSubcoreMesh(core_axis_name="c", subcore_axis_name="s")

    @functools.partial(
        pl.kernel,
        mesh=mesh,
        compiler_params=pltpu.CompilerParams(
            use_tc_tiling_on_sc=False, needs_layout_passes=False),
        out_type=jax.ShapeDtypeStruct((BATCH, EMB), jnp.float32),
        scratch_types=[
            pltpu.VMEM((SEQ, BPW), jnp.int32),
            pltpu.VMEM((NBUF, BPW, EMB), jnp.bfloat16),
            pltpu.VMEM((BPW, EMB), jnp.float32),
            pltpu.SemaphoreType.DMA,
            pltpu.SemaphoreType.DMA,
            pltpu.SemaphoreType.DMA,
            pltpu.SemaphoreType.DMA,
        ],
    )
    def pool_k(x_hbm, tab_hbm, out_hbm, idx_v, rows_v, acc_v, s0, s1, s2, s3):
        wid = lax.axis_index("s") * NC + lax.axis_index("c")
        base = wid * BPW
        pltpu.sync_copy(x_hbm.at[:, pl.ds(base, BPW)], idx_v)

        sems = (s0, s1, s2, s3)

        z = jnp.zeros((16,), jnp.float32)

        def zbody(j, _):
            for u in range(ACC_UNROLL):
                b = j * ACC_UNROLL + u
                acc_v[b, 0:16] = z
                acc_v[b, 16:32] = z
            return 0

        lax.fori_loop(0, BPW // ACC_UNROLL, zbody, 0)

        def fire(s, k):
            pltpu.async_copy(tab_hbm.at[idx_v.at[s]], rows_v.at[k], sems[k])

        def drain(s, k):
            pltpu.make_async_copy(tab_hbm.at[idx_v.at[s]], rows_v.at[k], sems[k]).wait()

        def accumulate(k):
            def body(j, _):
                for u in range(ACC_UNROLL):
                    b = j * ACC_UNROLL + u
                    lo, hi = plsc.unpack(
                        rows_v[k, b, 0:32], format=plsc.PackFormat.INTERLEAVED)
                    plsc.addupdate(acc_v.at[b, pl.ds(0, 16)], lo)
                    plsc.addupdate(acc_v.at[b, pl.ds(16, 16)], hi)
                return 0

            lax.fori_loop(0, BPW // ACC_UNROLL, body, 0)

        for k in range(NBUF):
            fire(k, k)

        def step(i, _):
            s = NBUF * i
            for k in range(NBUF):
                drain(s + k, k)
                accumulate(k)

                @pl.when(i < NSTEP - 1)
                def _(k=k, s=s):
                    fire(s + NBUF + k, k)

            return 0

        lax.fori_loop(0, NSTEP, step, 0)
        pltpu.sync_copy(acc_v, out_hbm.at[pl.ds(base, BPW)])

    return pool_k(xr, tab_lin)


def _mlp_tc(pooled_sum, age2, w1, wa, b1, w2, b2):
    def body(p_ref, age_ref, w1_ref, wa_ref, b1_ref, w2_ref, b2_ref, o_ref):
        pooled = p_ref[...] * jnp.float32(1.0 / SEQ)
        h = jnp.dot(pooled, w1_ref[...], preferred_element_type=jnp.float32)
        h = h + age_ref[...] * wa_ref[...] + b1_ref[...]
        logits = jnp.dot(h, w2_ref[...], preferred_element_type=jnp.float32)
        logits = logits + b2_ref[...]
        m = jnp.max(logits, axis=-1, keepdims=True)
        s = logits - m
        lse = jnp.log(jnp.sum(jnp.exp(s), axis=-1, keepdims=True))
        o_ref[...] = s - lse

    return pl.pallas_call(
        body,
        out_shape=jax.ShapeDtypeStruct((BATCH, OUT), jnp.float32),
    )(pooled_sum, age2, w1, wa, b1, w2, b2)


def kernel(x, age, emb_table, fc_w, fc_b, hid_w, hid_b):
    tT = jnp.transpose(emb_table)
    tab_pk = _pack_table_tc(tT)
    tab_lin = jnp.reshape(tab_pk, (VPAD, EMB))

    xi = x.astype(jnp.int32)
    xr = ((xi // TBLK) * TBLK) + ((xi % SUB) * 4) + ((xi // SUB) % 4)

    pooled_sum = _pooled_sum_sc(xr, tab_lin)

    age2 = age.reshape(BATCH, 1)
    perm = jnp.asarray(list(range(0, EMB, 2)) + list(range(1, EMB, 2)))
    w1 = fc_w[:, :EMB].T[perm]
    wa = fc_w[:, EMB:].T
    b1 = fc_b.reshape(1, HIDDEN)
    w2 = hid_w.T
    b2 = hid_b.reshape(1, OUT)
    return _mlp_tc(pooled_sum, age2, w1, wa, b1, w2, b2)


# speedup vs baseline: 2.3775x; 2.3775x over previous
import functools

import jax
import jax.numpy as jnp
from jax import lax
from jax.experimental import pallas as pl
from jax.experimental.pallas import tpu as pltpu
from jax.experimental.pallas import tpu_sc as plsc

VOCAB = 1000000
EMB = 32
HIDDEN = 50
OUT = 100
SEQ = 200
BATCH = 4096

NC = 2
NS = 16
NW = NC * NS
BPW = BATCH // NW
NBUF = 4
NSTEP = SEQ // NBUF
ACC_UNROLL = 8

TBLK = 32768
SUB = TBLK // 4
NTBLK = (VOCAB + TBLK - 1) // TBLK
VPAD = NTBLK * TBLK
PK_ROWS = VPAD // 4


def _pack_table_tc(tT):
    def body(t_ref, o_ref):
        xblk = t_ref[...]
        xs = jnp.concatenate(
            [xblk[:, a * SUB:(a + 1) * SUB] for a in range(4)], axis=0)
        eye = jnp.eye(128, dtype=jnp.float32)
        packed = lax.dot_general(
            xs, eye, (((0,), (0,)), ((), ())),
            preferred_element_type=jnp.float32)
        o_ref[...] = packed

    return pl.pallas_call(
        body,
        grid=(NTBLK,),
        in_specs=[pl.BlockSpec((EMB, TBLK), lambda i: (0, i))],
        out_specs=pl.BlockSpec((TBLK // 4, 128), lambda i: (i, 0)),
        out_shape=jax.ShapeDtypeStruct((PK_ROWS, 128), jnp.float32),
    )(tT)


def _pooled_sum_sc(xr, tab_lin):
    mesh = plsc.VectorSubcoreMesh(core_axis_name="c", subcore_axis_name="s")

    @functools.partial(
        pl.kernel,
        mesh=mesh,
        compiler_params=pltpu.CompilerParams(use_tc_tiling_on_sc=False),
        out_type=jax.ShapeDtypeStruct((BATCH, EMB), jnp.float32),
        scratch_types=[
            pltpu.VMEM((SEQ, BPW), jnp.int32),
            pltpu.VMEM((NBUF, BPW, EMB), jnp.float32),
            pltpu.VMEM((BPW, EMB), jnp.float32),
            pltpu.SemaphoreType.DMA,
            pltpu.SemaphoreType.DMA,
            pltpu.SemaphoreType.DMA,
            pltpu.SemaphoreType.DMA,
        ],
    )
    def pool_k(x_hbm, tab_hbm, out_hbm, idx_v, rows_v, acc_v, s0, s1, s2, s3):
        wid = lax.axis_index("s") * NC + lax.axis_index("c")
        base = wid * BPW
        pltpu.sync_copy(x_hbm.at[:, pl.ds(base, BPW)], idx_v)

        sems = (s0, s1, s2, s3)

        z = jnp.zeros((16,), jnp.float32)

        def zbody(j, _):
            for u in range(ACC_UNROLL):
                b = j * ACC_UNROLL + u
                acc_v[b, 0:16] = z
                acc_v[b, 16:32] = z
            return 0

        lax.fori_loop(0, BPW // ACC_UNROLL, zbody, 0)

        def fire(s, k):
            pltpu.async_copy(tab_hbm.at[idx_v.at[s]], rows_v.at[k], sems[k])

        def drain(s, k):
            pltpu.make_async_copy(tab_hbm.at[idx_v.at[s]], rows_v.at[k], sems[k]).wait()

        def accumulate(k):
            def body(j, _):
                for u in range(ACC_UNROLL):
                    b = j * ACC_UNROLL + u
                    plsc.addupdate(acc_v.at[b, pl.ds(0, 16)], rows_v[k, b, 0:16])
                    plsc.addupdate(acc_v.at[b, pl.ds(16, 16)], rows_v[k, b, 16:32])
                return 0

            lax.fori_loop(0, BPW // ACC_UNROLL, body, 0)

        for k in range(NBUF):
            fire(k, k)

        def step(i, _):
            s = NBUF * i
            for k in range(NBUF):
                drain(s + k, k)
                accumulate(k)

                @pl.when(i < NSTEP - 1)
                def _(k=k, s=s):
                    fire(s + NBUF + k, k)

            return 0

        lax.fori_loop(0, NSTEP, step, 0)
        pltpu.sync_copy(acc_v, out_hbm.at[pl.ds(base, BPW)])

    return pool_k(xr, tab_lin)


def _mlp_tc(pooled_sum, age2, w1, wa, b1, w2, b2):
    def body(p_ref, age_ref, w1_ref, wa_ref, b1_ref, w2_ref, b2_ref, o_ref):
        pooled = p_ref[...] * jnp.float32(1.0 / SEQ)
        h = jnp.dot(pooled, w1_ref[...], preferred_element_type=jnp.float32)
        h = h + age_ref[...] * wa_ref[...] + b1_ref[...]
        logits = jnp.dot(h, w2_ref[...], preferred_element_type=jnp.float32)
        logits = logits + b2_ref[...]
        m = jnp.max(logits, axis=-1, keepdims=True)
        s = logits - m
        lse = jnp.log(jnp.sum(jnp.exp(s), axis=-1, keepdims=True))
        o_ref[...] = s - lse

    return pl.pallas_call(
        body,
        out_shape=jax.ShapeDtypeStruct((BATCH, OUT), jnp.float32),
    )(pooled_sum, age2, w1, wa, b1, w2, b2)


def kernel(x, age, emb_table, fc_w, fc_b, hid_w, hid_b):
    tT = jnp.transpose(emb_table)
    tab_pk = _pack_table_tc(tT)
    tab_lin = jnp.reshape(tab_pk, (VPAD, EMB))

    xi = x.astype(jnp.int32)
    xr = ((xi // TBLK) * TBLK) + ((xi % SUB) * 4) + ((xi // SUB) % 4)

    pooled_sum = _pooled_sum_sc(xr, tab_lin)

    age2 = age.reshape(BATCH, 1)
    w1 = fc_w[:, :EMB].T
    wa = fc_w[:, EMB:].T
    b1 = fc_b.reshape(1, HIDDEN)
    w2 = hid_w.T
    b2 = hid_b.reshape(1, OUT)
    return _mlp_tc(pooled_sum, age2, w1, wa, b1, w2, b2)


# speedup vs baseline: 2.3852x; 1.0032x over previous
import functools

import jax
import jax.numpy as jnp
from jax import lax
from jax.experimental import pallas as pl
from jax.experimental.pallas import tpu as pltpu
from jax.experimental.pallas import tpu_sc as plsc

VOCAB = 1000000
EMB = 32
HIDDEN = 50
OUT = 100
SEQ = 200
BATCH = 4096

NC = 2
NS = 16
NW = NC * NS
BPW = BATCH // NW
NBUF = 4
NSTEP = SEQ // NBUF
ACC_UNROLL = 16

TBLK = 65536
SUB = TBLK // 4
NTBLK = (VOCAB + TBLK - 1) // TBLK
VPAD = NTBLK * TBLK
PK_ROWS = VPAD // 4


def _pack_table_tc(tT):
    def body(t_ref, o_ref):
        xblk = t_ref[...]
        xs = jnp.concatenate(
            [xblk[:, a * SUB:(a + 1) * SUB] for a in range(4)], axis=0)
        eye = jnp.eye(128, dtype=jnp.float32)
        packed = lax.dot_general(
            xs, eye, (((0,), (0,)), ((), ())),
            preferred_element_type=jnp.float32)
        o_ref[...] = packed

    return pl.pallas_call(
        body,
        grid=(NTBLK,),
        in_specs=[pl.BlockSpec((EMB, TBLK), lambda i: (0, i))],
        out_specs=pl.BlockSpec((TBLK // 4, 128), lambda i: (i, 0)),
        out_shape=jax.ShapeDtypeStruct((PK_ROWS, 128), jnp.float32),
    )(tT)


def _pooled_sum_sc(xr, tab_lin):
    mesh = plsc.VectorSubcoreMesh(core_axis_name="c", subcore_axis_name="s")

    @functools.partial(
        pl.kernel,
        mesh=mesh,
        compiler_params=pltpu.CompilerParams(use_tc_tiling_on_sc=False),
        out_type=jax.ShapeDtypeStruct((BATCH, EMB), jnp.float32),
        scratch_types=[
            pltpu.VMEM((SEQ, BPW), jnp.int32),
            pltpu.VMEM((NBUF, BPW, EMB), jnp.float32),
            pltpu.VMEM((BPW, EMB), jnp.float32),
            pltpu.SemaphoreType.DMA,
            pltpu.SemaphoreType.DMA,
            pltpu.SemaphoreType.DMA,
            pltpu.SemaphoreType.DMA,
        ],
    )
    def pool_k(x_hbm, tab_hbm, out_hbm, idx_v, rows_v, acc_v, s0, s1, s2, s3):
        wid = lax.axis_index("s") * NC + lax.axis_index("c")
        base = wid * BPW
        pltpu.sync_copy(x_hbm.at[:, pl.ds(base, BPW)], idx_v)

        sems = (s0, s1, s2, s3)

        z = jnp.zeros((16,), jnp.float32)

        def zbody(j, _):
            for u in range(ACC_UNROLL):
                b = j * ACC_UNROLL + u
                acc_v[b, 0:16] = z
                acc_v[b, 16:32] = z
            return 0

        lax.fori_loop(0, BPW // ACC_UNROLL, zbody, 0)

        def fire(s, k):
            pltpu.async_copy(tab_hbm.at[idx_v.at[s]], rows_v.at[k], sems[k])

        def drain(s, k):
            pltpu.make_async_copy(tab_hbm.at[idx_v.at[s]], rows_v.at[k], sems[k]).wait()

        def accumulate(k):
            def body(j, _):
                for u in range(ACC_UNROLL):
                    b = j * ACC_UNROLL + u
                    plsc.addupdate(acc_v.at[b, pl.ds(0, 16)], rows_v[k, b, 0:16])
                    plsc.addupdate(acc_v.at[b, pl.ds(16, 16)], rows_v[k, b, 16:32])
                return 0

            lax.fori_loop(0, BPW // ACC_UNROLL, body, 0)

        for k in range(NBUF):
            fire(k, k)

        def step(i, _):
            s = NBUF * i
            for k in range(NBUF):
                drain(s + k, k)
                accumulate(k)

                @pl.when(i < NSTEP - 1)
                def _(k=k, s=s):
                    fire(s + NBUF + k, k)

            return 0

        lax.fori_loop(0, NSTEP, step, 0)
        pltpu.sync_copy(acc_v, out_hbm.at[pl.ds(base, BPW)])

    return pool_k(xr, tab_lin)


def _mlp_tc(pooled_sum, age2, w1, wa, b1, w2, b2):
    def body(p_ref, age_ref, w1_ref, wa_ref, b1_ref, w2_ref, b2_ref, o_ref):
        pooled = p_ref[...] * jnp.float32(1.0 / SEQ)
        h = jnp.dot(pooled, w1_ref[...], preferred_element_type=jnp.float32)
        h = h + age_ref[...] * wa_ref[...] + b1_ref[...]
        logits = jnp.dot(h, w2_ref[...], preferred_element_type=jnp.float32)
        logits = logits + b2_ref[...]
        m = jnp.max(logits, axis=-1, keepdims=True)
        s = logits - m
        lse = jnp.log(jnp.sum(jnp.exp(s), axis=-1, keepdims=True))
        o_ref[...] = s - lse

    return pl.pallas_call(
        body,
        out_shape=jax.ShapeDtypeStruct((BATCH, OUT), jnp.float32),
    )(pooled_sum, age2, w1, wa, b1, w2, b2)


def kernel(x, age, emb_table, fc_w, fc_b, hid_w, hid_b):
    tT = jnp.transpose(emb_table)
    tab_pk = _pack_table_tc(tT)
    tab_lin = jnp.reshape(tab_pk, (VPAD, EMB))

    xi = x.astype(jnp.int32)
    xr = ((xi // TBLK) * TBLK) + ((xi % SUB) * 4) + ((xi // SUB) % 4)

    pooled_sum = _pooled_sum_sc(xr, tab_lin)

    age2 = age.reshape(BATCH, 1)
    w1 = fc_w[:, :EMB].T
    wa = fc_w[:, EMB:].T
    b1 = fc_b.reshape(1, HIDDEN)
    w2 = hid_w.T
    b2 = hid_b.reshape(1, OUT)
    return _mlp_tc(pooled_sum, age2, w1, wa, b1, w2, b2)
